# Initial kernel scaffold; baseline (speedup 1.0000x reference)
import jax
import jax.numpy as jnp
from jax.experimental import pallas as pl


def kernel(hidden_states, gamma, beta):
    raise NotImplementedError("write your pallas kernel here")


# speedup vs baseline: 23.9158x; 23.9158x over previous
import functools

import jax
import jax.numpy as jnp
from jax.experimental import pallas as pl
from jax.experimental.pallas import tpu as pltpu

KEEP_RATE = 0.8
EPS = 1e-5


def _scores_body(x_ref, out_ref):
    x = x_ref[0]
    sumsq = jnp.sum(x * x, axis=-1)
    out_ref[0, 0, :] = jnp.sqrt(sumsq)


def _thresh_body(k, bits_ref, v_ref, m_ref):
    batch, s = bits_ref.shape

    for r in range(batch):
        row = bits_ref[r : r + 1, :]

        def val_step(i, t):
            cand = t | (jnp.int32(1) << (30 - i))
            cnt = jnp.sum((row >= cand).astype(jnp.int32))
            return jnp.where(cnt >= k, cand, t)

        v = jax.lax.fori_loop(0, 31, val_step, jnp.int32(0))

        gt = row > v
        eq = row == v
        needed = k - jnp.sum(gt.astype(jnp.int32))
        idx = jax.lax.broadcasted_iota(jnp.int32, (1, s), 1)

        def idx_step(i, m):
            cand = jnp.minimum(m | (jnp.int32(1) << (13 - i)), jnp.int32(s))
            cnt = jnp.sum((eq & (idx < cand)).astype(jnp.int32))
            return jnp.where(cnt <= needed, cand, m)

        m = jax.lax.fori_loop(0, 14, idx_step, jnp.int32(0))

        v_ref[r] = v
        m_ref[r] = m


def _ln_body(nblk, bs, x_ref, v_ref, m_ref, gamma_ref, beta_ref, out_ref):
    i = pl.program_id(0)
    b = i // nblk
    x = x_ref[0]
    d = x.shape[-1]

    sumsq = jnp.sum(x * x, axis=-1, keepdims=True)
    bits = jax.lax.bitcast_convert_type(jnp.sqrt(sumsq), jnp.int32)
    tok = jax.lax.broadcasted_iota(jnp.int32, (bs, 1), 0) + (i % nblk) * bs
    v = v_ref[b]
    m = m_ref[b]
    keep = (bits > v) | ((bits == v) & (tok < m))

    mean = jnp.sum(x, axis=-1, keepdims=True) / d
    cent = x - mean
    var = jnp.sum(cent * cent, axis=-1, keepdims=True) / d
    ln = gamma_ref[...] * cent * jax.lax.rsqrt(var + EPS) + beta_ref[...]
    out_ref[0] = jnp.where(keep, ln, x)


@jax.jit
def kernel(hidden_states, gamma, beta):
    batch, seq, dim = hidden_states.shape
    keep_k = max(1, int(seq * KEEP_RATE))
    bs = min(1024, seq)
    nblk = seq // bs
    grid = batch * nblk

    scores3 = pl.pallas_call(
        _scores_body,
        grid=(grid,),
        in_specs=[
            pl.BlockSpec((1, bs, dim), lambda i: (i // nblk, i % nblk, 0)),
        ],
        out_specs=pl.BlockSpec((1, 1, bs), lambda i: (i, 0, 0)),
        out_shape=jax.ShapeDtypeStruct((grid, 1, bs), jnp.float32),
        compiler_params=pltpu.CompilerParams(
            dimension_semantics=("arbitrary",),
        ),
    )(hidden_states)
    scores = scores3.reshape(batch, seq)

    bits = jax.lax.bitcast_convert_type(scores, jnp.int32)

    v_arr, m_arr = pl.pallas_call(
        functools.partial(_thresh_body, keep_k),
        in_specs=[pl.BlockSpec(memory_space=pltpu.VMEM)],
        out_specs=[
            pl.BlockSpec(memory_space=pltpu.SMEM),
            pl.BlockSpec(memory_space=pltpu.SMEM),
        ],
        out_shape=[
            jax.ShapeDtypeStruct((batch,), jnp.int32),
            jax.ShapeDtypeStruct((batch,), jnp.int32),
        ],
    )(bits)

    out = pl.pallas_call(
        functools.partial(_ln_body, nblk, bs),
        grid=(grid,),
        in_specs=[
            pl.BlockSpec((1, bs, dim), lambda i: (i // nblk, i % nblk, 0)),
            pl.BlockSpec(memory_space=pltpu.SMEM),
            pl.BlockSpec(memory_space=pltpu.SMEM),
            pl.BlockSpec((dim,), lambda i: (0,)),
            pl.BlockSpec((dim,), lambda i: (0,)),
        ],
        out_specs=pl.BlockSpec((1, bs, dim), lambda i: (i // nblk, i % nblk, 0)),
        out_shape=jax.ShapeDtypeStruct((batch, seq, dim), jnp.float32),
        compiler_params=pltpu.CompilerParams(
            dimension_semantics=("arbitrary",),
        ),
    )(hidden_states, v_arr, m_arr, gamma, beta)
    return out
